# baseline (speedup 1.0000x reference)
import jax
import jax.numpy as jnp
import numpy as np
from jax.experimental import pallas as pl

N_LEVELS = 16
F = 2
LOG2_HS = 19
BASE = 16
FINEST = 512
_b = np.exp((np.log(FINEST) - np.log(BASE)) / (N_LEVELS - 1))
RES = [int(BASE * _b ** i) for i in range(N_LEVELS)]
TSIZES = [min(2 ** LOG2_HS, r ** 3) for r in RES]
PRIMES = (1, 2654435761, 805459861)


def _hash(ix, iy, iz, tsize):
    c = (ix.astype(jnp.uint32) * jnp.uint32(PRIMES[0])
         + iy.astype(jnp.uint32) * jnp.uint32(PRIMES[1])
         + iz.astype(jnp.uint32) * jnp.uint32(PRIMES[2]))
    return (c % jnp.uint32(tsize)).astype(jnp.int32)


def kernel(x, tables):
    feats = []
    for lvl in range(N_LEVELS):
        res = RES[lvl]
        tsize = TSIZES[lvl]
        tbl = tables[lvl]
        pos = x * res
        pos0f = jnp.floor(pos)
        w = pos - pos0f
        p0 = jnp.mod(pos0f.astype(jnp.int32), res)
        p1 = jnp.mod(pos0f.astype(jnp.int32) + 1, res)
        c000 = _hash(p0[:, 0], p0[:, 1], p0[:, 2], tsize)
        c001 = _hash(p0[:, 0], p0[:, 1], p1[:, 2], tsize)
        c010 = _hash(p0[:, 0], p1[:, 1], p0[:, 2], tsize)
        c011 = _hash(p0[:, 0], p1[:, 1], p1[:, 2], tsize)
        c100 = _hash(p1[:, 0], p0[:, 1], p0[:, 2], tsize)
        c101 = _hash(p1[:, 0], p0[:, 1], p1[:, 2], tsize)
        c110 = _hash(p1[:, 0], p1[:, 1], p0[:, 2], tsize)
        c111 = _hash(p1[:, 0], p1[:, 1], p1[:, 2], tsize)
        f000 = jnp.take(tbl, c000, axis=0)
        f001 = jnp.take(tbl, c001, axis=0)
        f010 = jnp.take(tbl, c010, axis=0)
        f011 = jnp.take(tbl, c011, axis=0)
        f100 = jnp.take(tbl, c100, axis=0)
        f101 = jnp.take(tbl, c101, axis=0)
        f110 = jnp.take(tbl, c110, axis=0)
        f111 = jnp.take(tbl, c111, axis=0)
        wx = w[:, 0:1]
        wy = w[:, 1:2]
        wz = w[:, 2:3]
        fx00 = f000 * (1 - wx) + f100 * wx
        fx01 = f001 * (1 - wx) + f101 * wx
        fx10 = f010 * (1 - wx) + f110 * wx
        fx11 = f011 * (1 - wx) + f111 * wx
        fxy0 = fx00 * (1 - wy) + fx10 * wy
        fxy1 = fx01 * (1 - wy) + fx11 * wy
        fxyz = fxy0 * (1 - wz) + fxy1 * wz
        feats.append(fxyz)
    return jnp.concatenate(feats, axis=-1)


# speedup vs baseline: 80.5271x; 80.5271x over previous
import functools

import numpy as np
import jax
import jax.numpy as jnp
from jax import lax
from jax.experimental import pallas as pl
from jax.experimental.pallas import tpu as pltpu, tpu_sc as plsc

N_LEVELS = 16
F = 2
LOG2_HS = 19
BASE = 16
FINEST = 512
_b = np.exp((np.log(FINEST) - np.log(BASE)) / (N_LEVELS - 1))
RES = [int(BASE * _b ** i) for i in range(N_LEVELS)]
TSIZES = [min(2 ** LOG2_HS, r ** 3) for r in RES]
OFFS = np.concatenate([[0], np.cumsum(TSIZES)]).astype(np.int64)
TOT_ROWS = int(OFFS[-1])
P2 = np.int32(np.uint32(2654435761).astype(np.int64) - (1 << 32))
P3 = np.int32(805459861)

N_POINTS = 1048576
NW = 32
PPW = N_POINTS // NW
C = 512
NG = C // 16
NIDX = 16 * C
IDX_MINOR = 128
NDMA = NIDX // IDX_MINOR
NCHUNK = PPW // C

_f32 = jnp.float32
_i32 = jnp.int32


def _mod_u32(c, m):
    if m & (m - 1) == 0:
        return jnp.bitwise_and(c, np.int32(m - 1))
    cf = c.astype(_f32)
    cf = jnp.where(c < 0, cf + np.float32(2.0 ** 32), cf)
    q = (cf * np.float32(1.0 / m)).astype(_i32)
    r = c - q * np.int32(m)
    r = jnp.where(r < 0, r + np.int32(m), r)
    r = jnp.where(r >= np.int32(m), r - np.int32(m), r)
    return r


def _axis_coords(v, res):
    p = v * np.float32(res)
    t = p.astype(_i32)
    frac = p - t.astype(_f32)
    i0 = jnp.where(t >= np.int32(res), t - np.int32(res), t)
    i1 = i0 + 1
    i1 = jnp.where(i1 == np.int32(res), 0, i1)
    return i0, i1, frac


def _sc_body(xh_hbm, yh_hbm, zh_hbm, tcat_hbm, out_hbm,
             xs, ys, zs, wx, wy, wz, idx_v, rows_v, out_v, sem):
    wid = lax.axis_index("s") * 2 + lax.axis_index("c")

    def chunk_body(t, carry):
        base = wid * PPW + t * C
        pltpu.sync_copy(xh_hbm.at[pl.ds(base, C)], xs)
        pltpu.sync_copy(yh_hbm.at[pl.ds(base, C)], ys)
        pltpu.sync_copy(zh_hbm.at[pl.ds(base, C)], zs)

        for lvl in range(N_LEVELS):
            res = RES[lvl]
            tsize = TSIZES[lvl]
            eoff = np.int32(2 * OFFS[lvl])

            def p1_body(g, carry, res=res, tsize=tsize, eoff=eoff):
                s = g * 16
                ix0, ix1, fx = _axis_coords(xs[pl.ds(s, 16)], res)
                iy0, iy1, fy = _axis_coords(ys[pl.ds(s, 16)], res)
                iz0, iz1, fz = _axis_coords(zs[pl.ds(s, 16)], res)
                wx[pl.ds(s, 16)] = fx
                wy[pl.ds(s, 16)] = fy
                wz[pl.ds(s, 16)] = fz
                hy0 = iy0 * P2
                hy1 = iy1 * P2
                hz0 = iz0 * P3
                hz1 = iz1 * P3
                s00 = ix0 + hy0
                s01 = ix1 + hy0
                s10 = ix0 + hy1
                s11 = ix1 + hy1
                corners = (s00 + hz0, s00 + hz1, s10 + hz0, s10 + hz1,
                           s01 + hz0, s01 + hz1, s11 + hz0, s11 + hz1)
                grow = g // 8
                gcol = (g % 8) * 16
                for k in range(8):
                    c = _mod_u32(corners[k], tsize)
                    e0 = c + c + eoff
                    idx_v[k * (2 * C // IDX_MINOR) + grow, pl.ds(gcol, 16)] = e0
                    idx_v[k * (2 * C // IDX_MINOR) + (C // IDX_MINOR) + grow,
                          pl.ds(gcol, 16)] = e0 + 1
                return carry

            lax.fori_loop(0, NG, p1_body, 0)

            def fire(j, carry):
                pltpu.make_async_copy(
                    tcat_hbm.at[idx_v.at[j]],
                    rows_v.at[pl.ds(j * IDX_MINOR, IDX_MINOR)],
                    sem).start()
                return carry

            lax.fori_loop(0, NDMA, fire, 0)

            def drain(j, carry):
                pltpu.make_async_copy(
                    tcat_hbm.at[idx_v.at[j]],
                    rows_v.at[pl.ds(j * IDX_MINOR, IDX_MINOR)],
                    sem).wait()
                return carry

            lax.fori_loop(0, NDMA, drain, 0)

            def p3_body(g, carry, lvl=lvl):
                s = g * 16
                fx = wx[pl.ds(s, 16)]
                fy = wy[pl.ds(s, 16)]
                fz = wz[pl.ds(s, 16)]
                gx = 1.0 - fx
                gy = 1.0 - fy
                gz = 1.0 - fz
                for j in range(F):
                    f = [rows_v[pl.ds(k * (2 * C) + j * C + s, 16)]
                         for k in range(8)]
                    a00 = f[0] * gx + f[4] * fx
                    a01 = f[1] * gx + f[5] * fx
                    a10 = f[2] * gx + f[6] * fx
                    a11 = f[3] * gx + f[7] * fx
                    b0 = a00 * gy + a10 * fy
                    b1 = a01 * gy + a11 * fy
                    out_v[2 * lvl + j, pl.ds(s, 16)] = b0 * gz + b1 * fz
                return carry

            lax.fori_loop(0, NG, p3_body, 0)

        pltpu.sync_copy(out_v, out_hbm.at[:, pl.ds(base, C)])
        return carry

    lax.fori_loop(0, NCHUNK, chunk_body, 0)


@jax.jit
def _encode_sc(xh, yh, zh, tcat):
    mesh = plsc.VectorSubcoreMesh(core_axis_name="c", subcore_axis_name="s",
                                  num_cores=2, num_subcores=16)
    f = pl.kernel(
        _sc_body,
        out_type=jax.ShapeDtypeStruct((N_LEVELS * F, N_POINTS), _f32),
        mesh=mesh,
        scratch_types=[
            pltpu.VMEM((C,), _f32),
            pltpu.VMEM((C,), _f32),
            pltpu.VMEM((C,), _f32),
            pltpu.VMEM((C,), _f32),
            pltpu.VMEM((C,), _f32),
            pltpu.VMEM((C,), _f32),
            pltpu.VMEM((NDMA, IDX_MINOR), _i32),
            pltpu.VMEM((NIDX,), _f32),
            pltpu.VMEM((N_LEVELS * F, C), _f32),
            pltpu.SemaphoreType.DMA,
        ],
    )
    return f(xh, yh, zh, tcat)


def kernel(x, tables):
    xh = x[:, 0]
    yh = x[:, 1]
    zh = x[:, 2]
    tcat = jnp.concatenate(tables, axis=0).reshape(-1)
    return _encode_sc(xh, yh, zh, tcat).T


# speedup vs baseline: 91.2136x; 1.1327x over previous
import functools

import numpy as np
import jax
import jax.numpy as jnp
from jax import lax
from jax.experimental import pallas as pl
from jax.experimental.pallas import tpu as pltpu, tpu_sc as plsc

N_LEVELS = 16
F = 2
LOG2_HS = 19
BASE = 16
FINEST = 512
_b = np.exp((np.log(FINEST) - np.log(BASE)) / (N_LEVELS - 1))
RES = [int(BASE * _b ** i) for i in range(N_LEVELS)]
TSIZES = [min(2 ** LOG2_HS, r ** 3) for r in RES]
P2 = np.int32(np.uint32(2654435761).astype(np.int64) - (1 << 32))
P3 = np.int32(805459861)

N_POINTS = 1048576
NW = 32
PPW = N_POINTS // NW
C = 512
NG = C // 16
NIDX = 16 * C
IDX_MINOR = 128
NDMA = NIDX // IDX_MINOR
NXD = 3 * C // IDX_MINOR
NCHUNK = PPW // C

_f32 = jnp.float32
_i32 = jnp.int32


def _mod_u32(c, m):
    if m & (m - 1) == 0:
        return jnp.bitwise_and(c, np.int32(m - 1))
    cf = c.astype(_f32)
    cf = jnp.where(c < 0, cf + np.float32(2.0 ** 32), cf)
    q = (cf * np.float32(1.0 / m)).astype(_i32)
    r = c - q * np.int32(m)
    r = jnp.where(r < 0, r + np.int32(m), r)
    r = jnp.where(r >= np.int32(m), r - np.int32(m), r)
    return r


def _axis_coords(v, res):
    p = v * np.float32(res)
    t = p.astype(_i32)
    frac = p - t.astype(_f32)
    i0 = jnp.where(t >= np.int32(res), t - np.int32(res), t)
    i1 = i0 + 1
    i1 = jnp.where(i1 == np.int32(res), 0, i1)
    return i0, i1, frac


def _sc_body(x_hbm, *rest):
    tbls = rest[:N_LEVELS]
    out_hbm = rest[N_LEVELS]
    (xs, ys, zs, wx, wy, wz, xidx_v, idx_v, rows_v, out_v, sem) = \
        rest[N_LEVELS + 1:]
    wid = lax.axis_index("s") * 2 + lax.axis_index("c")
    iot = lax.iota(_i32, 16)

    def chunk_body(t, carry):
        base = wid * PPW + t * C

        def xidx_body(g, carry):
            s = g * 16
            e = (base + s + iot) * 3
            grow = g // 8
            gcol = (g % 8) * 16
            for a in range(3):
                xidx_v[a * (C // IDX_MINOR) + grow, pl.ds(gcol, 16)] = e + a
            return carry

        lax.fori_loop(0, NG, xidx_body, 0)
        axes_v = (xs, ys, zs)
        for j in range(NXD):
            a, jj = j // (C // IDX_MINOR), j % (C // IDX_MINOR)
            pltpu.make_async_copy(
                x_hbm.at[xidx_v.at[j]],
                axes_v[a].at[pl.ds(jj * IDX_MINOR, IDX_MINOR)],
                sem).start()
        for j in range(NXD):
            a, jj = j // (C // IDX_MINOR), j % (C // IDX_MINOR)
            pltpu.make_async_copy(
                x_hbm.at[xidx_v.at[j]],
                axes_v[a].at[pl.ds(jj * IDX_MINOR, IDX_MINOR)],
                sem).wait()

        for lvl in range(N_LEVELS):
            res = RES[lvl]
            tsize = TSIZES[lvl]
            tbl = tbls[lvl]

            def p1_body(g, carry, res=res, tsize=tsize):
                s = g * 16
                ix0, ix1, fx = _axis_coords(xs[pl.ds(s, 16)], res)
                iy0, iy1, fy = _axis_coords(ys[pl.ds(s, 16)], res)
                iz0, iz1, fz = _axis_coords(zs[pl.ds(s, 16)], res)
                wx[pl.ds(s, 16)] = fx
                wy[pl.ds(s, 16)] = fy
                wz[pl.ds(s, 16)] = fz
                hy0 = iy0 * P2
                hy1 = iy1 * P2
                hz0 = iz0 * P3
                hz1 = iz1 * P3
                s00 = ix0 + hy0
                s01 = ix1 + hy0
                s10 = ix0 + hy1
                s11 = ix1 + hy1
                corners = (s00 + hz0, s00 + hz1, s10 + hz0, s10 + hz1,
                           s01 + hz0, s01 + hz1, s11 + hz0, s11 + hz1)
                grow = g // 8
                gcol = (g % 8) * 16
                for k in range(8):
                    c = _mod_u32(corners[k], tsize)
                    e0 = c + c
                    idx_v[k * (2 * C // IDX_MINOR) + grow, pl.ds(gcol, 16)] = e0
                    idx_v[k * (2 * C // IDX_MINOR) + (C // IDX_MINOR) + grow,
                          pl.ds(gcol, 16)] = e0 + 1
                return carry

            lax.fori_loop(0, NG, p1_body, 0)

            def fire(j, carry, tbl=tbl):
                pltpu.make_async_copy(
                    tbl.at[idx_v.at[j]],
                    rows_v.at[pl.ds(j * IDX_MINOR, IDX_MINOR)],
                    sem).start()
                return carry

            lax.fori_loop(0, NDMA, fire, 0)

            def drain(j, carry, tbl=tbl):
                pltpu.make_async_copy(
                    tbl.at[idx_v.at[j]],
                    rows_v.at[pl.ds(j * IDX_MINOR, IDX_MINOR)],
                    sem).wait()
                return carry

            lax.fori_loop(0, NDMA, drain, 0)

            def p3_body(g, carry, lvl=lvl):
                s = g * 16
                fx = wx[pl.ds(s, 16)]
                fy = wy[pl.ds(s, 16)]
                fz = wz[pl.ds(s, 16)]
                gx = 1.0 - fx
                gy = 1.0 - fy
                gz = 1.0 - fz
                for j in range(F):
                    f = [rows_v[pl.ds(k * (2 * C) + j * C + s, 16)]
                         for k in range(8)]
                    a00 = f[0] * gx + f[4] * fx
                    a01 = f[1] * gx + f[5] * fx
                    a10 = f[2] * gx + f[6] * fx
                    a11 = f[3] * gx + f[7] * fx
                    b0 = a00 * gy + a10 * fy
                    b1 = a01 * gy + a11 * fy
                    out_v[2 * lvl + j, pl.ds(s, 16)] = b0 * gz + b1 * fz
                return carry

            lax.fori_loop(0, NG, p3_body, 0)

        pltpu.sync_copy(out_v, out_hbm.at[:, pl.ds(base, C)])
        return carry

    lax.fori_loop(0, NCHUNK, chunk_body, 0)


TBLK = 512


def _tc_transpose_body(i_ref, o_ref):
    o_ref[...] = i_ref[...].T


@jax.jit
def _encode_sc(x_flat, *tbls):
    mesh = plsc.VectorSubcoreMesh(core_axis_name="c", subcore_axis_name="s",
                                  num_cores=2, num_subcores=16)
    f = pl.kernel(
        _sc_body,
        out_type=jax.ShapeDtypeStruct((N_LEVELS * F, N_POINTS), _f32),
        mesh=mesh,
        scratch_types=[
            pltpu.VMEM((C,), _f32),
            pltpu.VMEM((C,), _f32),
            pltpu.VMEM((C,), _f32),
            pltpu.VMEM((C,), _f32),
            pltpu.VMEM((C,), _f32),
            pltpu.VMEM((C,), _f32),
            pltpu.VMEM((NXD, IDX_MINOR), _i32),
            pltpu.VMEM((NDMA, IDX_MINOR), _i32),
            pltpu.VMEM((NIDX,), _f32),
            pltpu.VMEM((N_LEVELS * F, C), _f32),
            pltpu.SemaphoreType.DMA,
        ],
    )
    out32 = f(x_flat, *tbls)
    return pl.pallas_call(
        _tc_transpose_body,
        grid=(N_POINTS // TBLK,),
        in_specs=[pl.BlockSpec((N_LEVELS * F, TBLK), lambda i: (0, i))],
        out_specs=pl.BlockSpec((TBLK, N_LEVELS * F), lambda i: (i, 0)),
        out_shape=jax.ShapeDtypeStruct((N_POINTS, N_LEVELS * F), _f32),
    )(out32)


def kernel(x, tables):
    return _encode_sc(x.reshape(-1), *(t.reshape(-1) for t in tables))
